# Initial kernel scaffold; baseline (speedup 1.0000x reference)
import jax
import jax.numpy as jnp
from jax.experimental import pallas as pl


def kernel(x, edge_index, edge_attr, edge_type, W, att_src, att_dst, W_edge, att_edge, bias, et_w, Wq, bq, Wk, bk, Wv, bv, Wo, bo, W1, b1, W2, b2):
    raise NotImplementedError("write your pallas kernel here")


# speedup vs baseline: 31.8136x; 31.8136x over previous
import functools

import jax
import jax.numpy as jnp
import numpy as np
from jax import lax
from jax.experimental import pallas as pl
from jax.experimental.pallas import tpu as pltpu
from jax.experimental.pallas import tpu_sc as plsc

N = 4096
E = 262144
D = 128
NET = 4
NL = 3
ED = 10

NC, NS = 2, 16
NW = NC * NS
EPT = E // NW
CHUNK = 128
NCHUNK = EPT // CHUNK
STRIPE = N // NS

HI = lax.Precision.HIGHEST
NEG = -1e30

_EB = 8192


def _edges_body(ea_ref, et_ref, etr_ref, evec_ref, escT_ref, asum_ref,
                cnt_ref, cmax_ref):
    ea = ea_ref[...]
    et = et_ref[...]
    etr = etr_ref[...]
    escT = lax.dot_general(evec_ref[...], ea, (((1,), (1,)), ((), ())),
                           precision=HI)
    tmap = lax.broadcasted_iota(jnp.int32, (12, 1), 0) // NL
    escM = jnp.where(tmap == etr, escT, NEG)
    escT_ref[...] = escM
    oh = (et == lax.broadcasted_iota(jnp.int32, (1, NET), 1)).astype(jnp.float32)
    asum = lax.dot_general(oh, ea, (((0,), (0,)), ((), ())), precision=HI)
    cnt = jnp.sum(oh, axis=0, keepdims=True)
    cmax = jnp.max(escM, axis=1, keepdims=True)
    pi = pl.program_id(0)

    @pl.when(pi == 0)
    def _():
        asum_ref[...] = asum
        cnt_ref[...] = cnt
        cmax_ref[...] = cmax

    @pl.when(pi != 0)
    def _():
        asum_ref[...] += asum
        cnt_ref[...] += cnt
        cmax_ref[...] = jnp.maximum(cmax_ref[...], cmax)


def _prep_edges(ea, et2, etr, evec_r):
    grid = E // _EB
    return pl.pallas_call(
        _edges_body,
        grid=(grid,),
        in_specs=[
            pl.BlockSpec((_EB, ED), lambda i: (i, 0)),
            pl.BlockSpec((_EB, 1), lambda i: (i, 0)),
            pl.BlockSpec((1, _EB), lambda i: (0, i)),
            pl.BlockSpec((NET * NL, ED), lambda i: (0, 0)),
        ],
        out_specs=[
            pl.BlockSpec((NET * NL, _EB), lambda i: (0, i)),
            pl.BlockSpec((NET, ED), lambda i: (0, 0)),
            pl.BlockSpec((1, NET), lambda i: (0, 0)),
            pl.BlockSpec((NET * NL, 1), lambda i: (0, 0)),
        ],
        out_shape=[
            jax.ShapeDtypeStruct((NET * NL, E), jnp.float32),
            jax.ShapeDtypeStruct((NET, ED), jnp.float32),
            jax.ShapeDtypeStruct((1, NET), jnp.float32),
            jax.ShapeDtypeStruct((NET * NL, 1), jnp.float32),
        ],
    )(ea, et2, etr, evec_r)


def _layer_prep_body(h_ref, w_ref, as_ref, ad_ref, mesc_ref, xw_ref, ss_ref,
                     sd_ref, shift_ref):
    xw = jnp.dot(h_ref[...], w_ref[...], precision=HI)
    xw_ref[...] = xw
    ss = jnp.dot(xw, as_ref[...], precision=HI)
    sd = jnp.dot(xw, ad_ref[...], precision=HI)
    ss_ref[...] = ss
    sd_ref[...] = sd
    m = mesc_ref[0, 0] + jnp.max(ss)
    z = sd + m
    shift_ref[...] = jnp.maximum(z, 0.2 * z)


def _layer_prep(h, w, a_s, a_d, mesc):
    return pl.pallas_call(
        _layer_prep_body,
        out_shape=[
            jax.ShapeDtypeStruct((N, D), jnp.float32),
            jax.ShapeDtypeStruct((N, 1), jnp.float32),
            jax.ShapeDtypeStruct((N, 1), jnp.float32),
            jax.ShapeDtypeStruct((N, 1), jnp.float32),
        ],
    )(h, w, a_s, a_d, mesc)


_sc_mesh = plsc.VectorSubcoreMesh(core_axis_name="c", subcore_axis_name="s")


@functools.partial(
    pl.kernel,
    out_type=(
        jax.ShapeDtypeStruct((NC, N, D), jnp.float32),
        jax.ShapeDtypeStruct((NC, N), jnp.float32),
    ),
    mesh=_sc_mesh,
    compiler_params=pltpu.CompilerParams(needs_layout_passes=False),
    scratch_types=[
        pltpu.VMEM((N,), jnp.float32),
        pltpu.VMEM((N,), jnp.float32),
        pltpu.VMEM((N,), jnp.float32),
        pltpu.VMEM((CHUNK,), jnp.int32),
        pltpu.VMEM((CHUNK,), jnp.int32),
        pltpu.VMEM((CHUNK,), jnp.float32),
        pltpu.VMEM((CHUNK,), jnp.float32),
        pltpu.VMEM((CHUNK, D), jnp.float32),
        pltpu.VMEM_SHARED((N, D), jnp.float32),
        pltpu.VMEM_SHARED((N,), jnp.float32),
    ],
)
def _sc_edge(src_hbm, dst_hbm, esc_hbm, ssrc_hbm, sdst_hbm, shift_hbm, xw_hbm,
             numer_out, den_out, ssrc_v, sdst_v, shift_v, src_i, dst_i, esc_v,
             ex_v, rows_v, numer_sp, den_sp):
    cid = lax.axis_index("c")
    sid = lax.axis_index("s")
    wid = sid * NC + cid

    pltpu.sync_copy(ssrc_hbm, ssrc_v)
    pltpu.sync_copy(sdst_hbm, sdst_v)
    pltpu.sync_copy(shift_hbm, shift_v)

    z16 = jnp.zeros((16,), jnp.float32)

    def _zex(i, c):
        ex_v[pl.ds(i * 16, 16)] = z16
        return c

    lax.fori_loop(0, CHUNK // 16, _zex, 0)

    def _zrow(i, c):
        for col in range(D // 16):
            rows_v[i, pl.ds(col * 16, 16)] = z16
        return c

    lax.fori_loop(0, CHUNK, _zrow, 0)

    for half in range(STRIPE // CHUNK):
        off = sid * STRIPE + half * CHUNK
        pltpu.sync_copy(ex_v, den_sp.at[pl.ds(off, CHUNK)])
        pltpu.sync_copy(rows_v, numer_sp.at[pl.ds(off, CHUNK)])
    plsc.subcore_barrier()

    def _chunk(ci, c):
        base = wid * EPT + ci * CHUNK
        pltpu.sync_copy(src_hbm.at[pl.ds(base, CHUNK)], src_i)
        pltpu.sync_copy(dst_hbm.at[pl.ds(base, CHUNK)], dst_i)
        pltpu.sync_copy(esc_hbm.at[pl.ds(base, CHUNK)], esc_v)

        def _ex16(j, cc):
            o = j * 16
            s16 = src_i[pl.ds(o, 16)]
            d16 = dst_i[pl.ds(o, 16)]
            a = plsc.load_gather(ssrc_v, [s16])
            b = plsc.load_gather(sdst_v, [d16])
            sh = plsc.load_gather(shift_v, [d16])
            zz = a + b + esc_v[pl.ds(o, 16)]
            al = jnp.maximum(zz, 0.2 * zz)
            ex_v[pl.ds(o, 16)] = jnp.exp(al - sh)
            return cc

        lax.fori_loop(0, CHUNK // 16, _ex16, 0)
        pltpu.sync_copy(ex_v, den_sp.at[dst_i], add=True)
        pltpu.sync_copy(xw_hbm.at[src_i], rows_v)

        def _scale(j, cc):
            sp = plsc.load_gather(ex_v, [jnp.full((16,), j, jnp.int32)])
            for col in range(D // 16):
                rows_v[j, pl.ds(col * 16, 16)] = (
                    rows_v[j, pl.ds(col * 16, 16)] * sp)
            return cc

        lax.fori_loop(0, CHUNK, _scale, 0)
        pltpu.sync_copy(rows_v, numer_sp.at[dst_i], add=True)
        return c

    lax.fori_loop(0, NCHUNK, _chunk, 0)
    plsc.subcore_barrier()

    off = sid * STRIPE
    pltpu.sync_copy(den_sp.at[pl.ds(off, STRIPE)],
                    den_out.at[cid, pl.ds(off, STRIPE)])
    pltpu.sync_copy(numer_sp.at[pl.ds(off, STRIPE)],
                    numer_out.at[cid, pl.ds(off, STRIPE)])


def _combine_body(numer_ref, den_ref, xw_ref, ss_ref, sd_ref, shift_ref,
                  escs_ref, b_ref, h_ref):
    numer = numer_ref[0] + numer_ref[1]
    ones = jnp.ones((NC, 1), jnp.float32)
    den = lax.dot_general(den_ref[...], ones, (((0,), (0,)), ((), ())),
                          precision=HI)
    zs = ss_ref[...] + sd_ref[...] + escs_ref[0, 0]
    als = jnp.maximum(zs, 0.2 * zs)
    exs = jnp.exp(als - shift_ref[...])
    numer = numer + exs * xw_ref[...]
    den = den + exs
    hh = numer / (den + 1e-16) + b_ref[...]
    h_ref[...] = jnp.where(hh > 0, hh, jnp.exp(jnp.minimum(hh, 0.0)) - 1.0)


def _combine(numer, den, xw, ss, sd, shift, escs, b):
    return pl.pallas_call(
        _combine_body,
        out_shape=jax.ShapeDtypeStruct((N, D), jnp.float32),
    )(numer, den, xw, ss, sd, shift, escs, b)


def _fusion_body(h0_ref, h1_ref, h2_ref, h3_ref, etw_ref, wq_ref, bq_ref,
                 wk_ref, bk_ref, wv_ref, bv_ref, hsum_ref, q_ref, k_ref,
                 v_ref):
    e = etw_ref[...]
    e = e - jnp.max(e)
    e = jnp.exp(e)
    e = e / jnp.sum(e)
    hs = (e[0, 0] * h0_ref[...] + e[0, 1] * h1_ref[...] +
          e[0, 2] * h2_ref[...] + e[0, 3] * h3_ref[...])
    hsum_ref[...] = hs
    q_ref[...] = jnp.dot(hs, wq_ref[...], precision=HI) + bq_ref[...]
    k_ref[...] = jnp.dot(hs, wk_ref[...], precision=HI) + bk_ref[...]
    v_ref[...] = jnp.dot(hs, wv_ref[...], precision=HI) + bv_ref[...]


def _fusion(h0, h1, h2, h3, etw, wq, bq, wk, bk, wv, bv):
    return pl.pallas_call(
        _fusion_body,
        out_shape=[jax.ShapeDtypeStruct((N, D), jnp.float32)] * 4,
    )(h0, h1, h2, h3, etw, wq, bq, wk, bk, wv, bv)


_QB = 512
_DH = 32
_NH = 4


def _mha_body(q_ref, k_ref, v_ref, o_ref):
    q = q_ref[0]
    k = k_ref[0]
    v = v_ref[0]
    s = lax.dot_general(q, k, (((1,), (1,)), ((), ())))
    s = s * np.float32(1.0 / np.sqrt(_DH))
    m = jnp.max(s, axis=1, keepdims=True)
    p = jnp.exp(s - m)
    denom = jnp.sum(p, axis=1, keepdims=True)
    o = jnp.dot(p, v)
    o_ref[0] = o / denom


def _mha(q3, k3, v3):
    return pl.pallas_call(
        _mha_body,
        grid=(_NH, N // _QB),
        in_specs=[
            pl.BlockSpec((1, _QB, _DH), lambda h, i: (h, i, 0)),
            pl.BlockSpec((1, N, _DH), lambda h, i: (h, 0, 0)),
            pl.BlockSpec((1, N, _DH), lambda h, i: (h, 0, 0)),
        ],
        out_specs=pl.BlockSpec((1, _QB, _DH), lambda h, i: (h, i, 0)),
        out_shape=jax.ShapeDtypeStruct((_NH, N, _DH), jnp.float32),
    )(q3, k3, v3)


def _final_body(ao_ref, wo_ref, bo_ref, hsum_ref, w1_ref, b1_ref, w2_ref,
                b2_ref, out_ref):
    attn = jnp.dot(ao_ref[...], wo_ref[...], precision=HI) + bo_ref[...]
    ap = jnp.mean(attn, axis=0, keepdims=True)
    hs = hsum_ref[...]
    mp = jnp.mean(hs, axis=0, keepdims=True)
    xp = jnp.max(hs, axis=0, keepdims=True)
    g = (jnp.dot(mp, w1_ref[0:D], precision=HI) +
         jnp.dot(xp, w1_ref[D:2 * D], precision=HI) +
         jnp.dot(ap, w1_ref[2 * D:3 * D], precision=HI) + b1_ref[...])
    g = jnp.maximum(g, 0.0)
    o = jnp.dot(g, w2_ref[...], precision=HI) + b2_ref[...]
    out_ref[...] = jnp.maximum(o, 0.0)


def _final(ao, wo, bo, hsum, w1, b1, w2, b2):
    return pl.pallas_call(
        _final_body,
        out_shape=jax.ShapeDtypeStruct((1, 2 * D), jnp.float32),
    )(ao, wo, bo, hsum, w1, b1, w2, b2)


def kernel(x, edge_index, edge_attr, edge_type, W, att_src, att_dst, W_edge,
           att_edge, bias, et_w, Wq, bq, Wk, bk, Wv, bv, Wo, bo, W1, b1, W2,
           b2):
    src = edge_index[0]
    dst = edge_index[1]
    et2 = edge_type.reshape(E, 1)
    etr = edge_type.reshape(1, E)

    evec = jnp.einsum('tldh,tlh->tld', W_edge, att_edge)
    evec_r = evec.reshape(NET * NL, ED)
    escT, asum, cnt, cmax = _prep_edges(edge_attr, et2, etr, evec_r)
    mean_attr = asum / cnt.reshape(NET, 1)
    esc_self = jnp.einsum('td,tld->tl', mean_attr, evec)
    mesc = jnp.maximum(cmax.reshape(NET, NL), esc_self)

    outs = []
    for t in range(NET):
        h = x
        for l in range(NL):
            j = NL * t + l
            xw, ss, sd, shift = _layer_prep(
                h, W[t, l], att_src[t, l].reshape(D, 1),
                att_dst[t, l].reshape(D, 1), mesc[t, l].reshape(1, 1))
            numer, den = _sc_edge(src, dst, escT[j], ss.reshape(N),
                                  sd.reshape(N), shift.reshape(N), xw)
            h = _combine(numer, den, xw, ss, sd, shift,
                         esc_self[t, l].reshape(1, 1), bias[t, l].reshape(1, D))
        outs.append(h)

    hsum, q, k, v = _fusion(outs[0], outs[1], outs[2], outs[3],
                            et_w.reshape(1, NET), Wq, bq.reshape(1, D), Wk,
                            bk.reshape(1, D), Wv, bv.reshape(1, D))
    q3 = q.reshape(N, _NH, _DH).transpose(1, 0, 2)
    k3 = k.reshape(N, _NH, _DH).transpose(1, 0, 2)
    v3 = v.reshape(N, _NH, _DH).transpose(1, 0, 2)
    o3 = _mha(q3, k3, v3)
    ao = o3.transpose(1, 0, 2).reshape(N, D)
    out = _final(ao, Wo, bo.reshape(1, D), hsum, W1, b1.reshape(1, 2 * D),
                 W2, b2.reshape(1, 2 * D))
    return out
